# baseline (speedup 1.0000x reference)
import functools

import jax
import jax.numpy as jnp
from jax import lax
from jax.experimental import pallas as pl
from jax.experimental.pallas import tpu as pltpu
from jax.experimental.pallas import tpu_sc as plsc

NC = 2
NS = 16
NW = NC * NS
C = 128
NBUF = 3
HQ = 4


@functools.lru_cache(maxsize=None)
def _make_gather(V, D, B):
    assert B % C == 0 and B % 8 == 0
    n_chunks = B // C
    per_w = (n_chunks + NW - 1) // NW
    rounds = (per_w + 1 + NBUF - 1) // NBUF
    mesh = plsc.VectorSubcoreMesh(core_axis_name="c", subcore_axis_name="s")

    v_blk = ((V + NS - 1) // NS + 7) // 8 * 8
    stage_blocks = []
    off = 0
    while off < V:
        stage_blocks.append((off, min(v_blk, V - off)))
        off += v_blk

    scratch = (
        [pltpu.VMEM((C,), jnp.int32) for _ in range(NBUF)]
        + [pltpu.VMEM((C, D), jnp.float32) for _ in range(NBUF)]
        + [pltpu.SemaphoreType.DMA] * (3 * NBUF)
        + [pltpu.VMEM_SHARED((V, D), jnp.float32), pltpu.SemaphoreType.DMA]
    )

    @functools.partial(
        pl.kernel,
        out_type=jax.ShapeDtypeStruct((B, D), jnp.float32),
        mesh=mesh,
        scratch_types=scratch,
    )
    def k(x_hbm, idx_hbm, out_hbm, *bufs):
        idx_v = bufs[:NBUF]
        rows = bufs[NBUF:2 * NBUF]
        idx_sem = bufs[2 * NBUF:3 * NBUF]
        g_sem = bufs[3 * NBUF:4 * NBUF]
        s_sem = bufs[4 * NBUF:5 * NBUF]
        xs = bufs[5 * NBUF]
        stage_sem = bufs[5 * NBUF + 1]
        sid = lax.axis_index("s")
        wid = sid * NC + lax.axis_index("c")

        for t, (boff, blen) in enumerate(stage_blocks):
            @pl.when(sid == t)
            def _(boff=boff, blen=blen):
                pltpu.async_copy(x_hbm.at[pl.ds(boff, blen)],
                                 xs.at[pl.ds(boff, blen)], stage_sem)

        for j in range(NBUF):
            c0 = j * NW + wid

            @pl.when(c0 < n_chunks)
            def _(j=j, c0=c0):
                pltpu.async_copy(idx_hbm.at[pl.ds(c0 * C, C)], idx_v[j],
                                 idx_sem[j])

        for t, (boff, blen) in enumerate(stage_blocks):
            @pl.when(sid == t)
            def _(boff=boff, blen=blen):
                pltpu.make_async_copy(x_hbm.at[pl.ds(boff, blen)],
                                      xs.at[pl.ds(boff, blen)],
                                      stage_sem).wait()
        plsc.subcore_barrier()

        def round_body(r, carry):
            for j in range(NBUF):
                i = r * NBUF + j
                c = i * NW + wid

                @pl.when(c < n_chunks)
                def _(j=j, i=i, c=c):
                    @pl.when(i >= NBUF)
                    def _():
                        pltpu.make_async_copy(
                            rows[j], out_hbm.at[pl.ds(0, C)], s_sem[j]
                        ).wait()

                    pltpu.make_async_copy(
                        idx_hbm.at[pl.ds(0, C)], idx_v[j], idx_sem[j]
                    ).wait()
                    use_hbm = lax.rem(i, HQ) == 0

                    @pl.when(use_hbm)
                    def _():
                        pltpu.async_copy(x_hbm.at[idx_v[j]], rows[j],
                                         g_sem[j])

                    @pl.when(jnp.logical_not(use_hbm))
                    def _():
                        pltpu.async_copy(xs.at[idx_v[j]], rows[j], g_sem[j])

                p = (j - 1) % NBUF
                ip = i - 1
                cp = ip * NW + wid

                @pl.when((ip >= 0) & (cp < n_chunks))
                def _(p=p, ip=ip, cp=cp):
                    use_hbm_p = lax.rem(ip, HQ) == 0

                    @pl.when(use_hbm_p)
                    def _():
                        pltpu.make_async_copy(
                            x_hbm.at[idx_v[p]], rows[p], g_sem[p]
                        ).wait()

                    @pl.when(jnp.logical_not(use_hbm_p))
                    def _():
                        pltpu.make_async_copy(
                            xs.at[idx_v[p]], rows[p], g_sem[p]
                        ).wait()
                    pltpu.async_copy(rows[p], out_hbm.at[pl.ds(cp * C, C)],
                                     s_sem[p])
                    cn = (ip + NBUF) * NW + wid

                    @pl.when(cn < n_chunks)
                    def _():
                        pltpu.async_copy(idx_hbm.at[pl.ds(cn * C, C)],
                                         idx_v[p], idx_sem[p])

            return carry

        lax.fori_loop(0, rounds, round_body, 0)

        for j in range(NBUF):
            pltpu.make_async_copy(
                rows[j], out_hbm.at[pl.ds(0, C)], s_sem[j]
            ).wait()

    return k


def kernel(x, neighbor_indices):
    idx = neighbor_indices.astype(jnp.int32)
    V, D = x.shape
    B = idx.shape[0]
    return _make_gather(V, D, B)(x, idx)


# speedup vs baseline: 1.1732x; 1.1732x over previous
import functools

import jax
import jax.numpy as jnp
from jax import lax
from jax.experimental import pallas as pl
from jax.experimental.pallas import tpu as pltpu
from jax.experimental.pallas import tpu_sc as plsc

NC = 2
NS = 16
NW = NC * NS
C = 64
NBUF = 6


@functools.lru_cache(maxsize=None)
def _make_gather(V, D, B):
    assert B % C == 0 and B % 8 == 0
    n_chunks = B // C
    per_w = (n_chunks + NW - 1) // NW
    rounds = (per_w + 1 + NBUF - 1) // NBUF
    assert per_w >= NBUF
    mesh = plsc.VectorSubcoreMesh(core_axis_name="c", subcore_axis_name="s")

    v_blk = ((V + NS - 1) // NS + 7) // 8 * 8
    stage_blocks = []
    off = 0
    while off < V:
        stage_blocks.append((off, min(v_blk, V - off)))
        off += v_blk

    scratch = (
        [pltpu.VMEM((C,), jnp.int32) for _ in range(NBUF)]
        + [pltpu.VMEM((C, D), jnp.float32) for _ in range(NBUF)]
        + [pltpu.SemaphoreType.DMA] * (3 * NBUF)
        + [pltpu.VMEM_SHARED((V, D), jnp.float32), pltpu.SemaphoreType.DMA]
    )

    @functools.partial(
        pl.kernel,
        out_type=jax.ShapeDtypeStruct((B, D), jnp.float32),
        mesh=mesh,
        scratch_types=scratch,
    )
    def k(x_hbm, idx_hbm, out_hbm, *bufs):
        idx_v = bufs[:NBUF]
        rows = bufs[NBUF:2 * NBUF]
        idx_sem = bufs[2 * NBUF:3 * NBUF]
        g_sem = bufs[3 * NBUF:4 * NBUF]
        s_sem = bufs[4 * NBUF:5 * NBUF]
        xs = bufs[5 * NBUF]
        stage_sem = bufs[5 * NBUF + 1]
        sid = lax.axis_index("s")
        wid = sid * NC + lax.axis_index("c")

        for t, (boff, blen) in enumerate(stage_blocks):
            @pl.when(sid == t)
            def _(boff=boff, blen=blen):
                pltpu.async_copy(x_hbm.at[pl.ds(boff, blen)],
                                 xs.at[pl.ds(boff, blen)], stage_sem)

        for j in range(NBUF):
            c0 = j * NW + wid

            @pl.when(c0 < n_chunks)
            def _(j=j, c0=c0):
                pltpu.async_copy(idx_hbm.at[pl.ds(c0 * C, C)], idx_v[j],
                                 idx_sem[j])

        for t, (boff, blen) in enumerate(stage_blocks):
            @pl.when(sid == t)
            def _(boff=boff, blen=blen):
                pltpu.make_async_copy(x_hbm.at[pl.ds(boff, blen)],
                                      xs.at[pl.ds(boff, blen)],
                                      stage_sem).wait()
        plsc.subcore_barrier()

        for j in range(NBUF):
            c0 = j * NW + wid

            @pl.when(c0 < n_chunks)
            def _(j=j):
                pltpu.make_async_copy(
                    idx_hbm.at[pl.ds(0, C)], idx_v[j], idx_sem[j]
                ).wait()
                pltpu.async_copy(xs.at[idx_v[j]], rows[j], g_sem[j])

            if j >= 1:
                p = j - 1
                cp = p * NW + wid

                @pl.when(cp < n_chunks)
                def _(p=p, cp=cp):
                    pltpu.make_async_copy(
                        xs.at[idx_v[p]], rows[p], g_sem[p]
                    ).wait()
                    pltpu.async_copy(rows[p], out_hbm.at[pl.ds(cp * C, C)],
                                     s_sem[p])
                    cn = (p + NBUF) * NW + wid

                    @pl.when(cn < n_chunks)
                    def _():
                        pltpu.async_copy(idx_hbm.at[pl.ds(cn * C, C)],
                                         idx_v[p], idx_sem[p])

        def round_body(r, carry):
            for j in range(NBUF):
                i = r * NBUF + j
                c = i * NW + wid

                @pl.when(c < n_chunks)
                def _(j=j, c=c):
                    pltpu.make_async_copy(
                        rows[j], out_hbm.at[pl.ds(0, C)], s_sem[j]
                    ).wait()
                    pltpu.make_async_copy(
                        idx_hbm.at[pl.ds(0, C)], idx_v[j], idx_sem[j]
                    ).wait()
                    pltpu.async_copy(xs.at[idx_v[j]], rows[j], g_sem[j])

                p = (j - 1) % NBUF
                cp = (i - 1) * NW + wid

                @pl.when(cp < n_chunks)
                def _(p=p, i=i, cp=cp):
                    pltpu.make_async_copy(
                        xs.at[idx_v[p]], rows[p], g_sem[p]
                    ).wait()
                    pltpu.async_copy(rows[p], out_hbm.at[pl.ds(cp * C, C)],
                                     s_sem[p])
                    cn = (i - 1 + NBUF) * NW + wid

                    @pl.when(cn < n_chunks)
                    def _():
                        pltpu.async_copy(idx_hbm.at[pl.ds(cn * C, C)],
                                         idx_v[p], idx_sem[p])

            return carry

        lax.fori_loop(1, rounds, round_body, 0)

        for j in range(NBUF):
            pltpu.make_async_copy(
                rows[j], out_hbm.at[pl.ds(0, C)], s_sem[j]
            ).wait()

    return k


def kernel(x, neighbor_indices):
    idx = neighbor_indices.astype(jnp.int32)
    V, D = x.shape
    B = idx.shape[0]
    return _make_gather(V, D, B)(x, idx)
